# initial kernel scaffold (unmeasured)
import jax
import jax.numpy as jnp
from jax import lax
from jax.experimental import pallas as pl
from jax.experimental.pallas import tpu as pltpu

N_DEV = 8


def kernel(x, w_mat):
    m, k_local = x.shape
    _, n = w_mat.shape
    m_chunk = m // N_DEV

    def body(x_hbm, w_ref, out_hbm, xbuf, comm, load_sem, store_sem,
             send_sems, recv_sems, credit_sem):
        my = lax.axis_index("i")
        left = lax.rem(my - 1 + N_DEV, N_DEV)
        right = lax.rem(my + 1, N_DEV)

        barrier = pltpu.get_barrier_semaphore()
        for nbr in (left, right):
            pl.semaphore_signal(
                barrier, inc=1, device_id=(nbr,),
                device_id_type=pl.DeviceIdType.MESH,
            )
        pl.semaphore_wait(barrier, 2)

        def load_chunk(c):
            cp = pltpu.make_async_copy(
                x_hbm.at[pl.ds(c * m_chunk, m_chunk), :], xbuf, load_sem)
            cp.start()
            return cp

        def local_partial():
            return jnp.dot(
                xbuf[...], w_ref[...], preferred_element_type=jnp.float32)

        load_chunk(lax.rem(my - 1 + N_DEV, N_DEV)).wait()
        comm[0] = local_partial()

        for s in range(N_DEV - 1):
            a = s % 2
            b = (s + 1) % 2
            if s >= 1:
                pl.semaphore_wait(credit_sem, 1)
            rdma = pltpu.make_async_remote_copy(
                src_ref=comm.at[a],
                dst_ref=comm.at[b],
                send_sem=send_sems.at[a],
                recv_sem=recv_sems.at[b],
                device_id=(right,),
                device_id_type=pl.DeviceIdType.MESH,
            )
            rdma.start()
            c = lax.rem(my - s - 2 + 2 * N_DEV, N_DEV)
            xcp = load_chunk(c)
            rdma.wait()
            if s < N_DEV - 2:
                pl.semaphore_signal(
                    credit_sem, inc=1, device_id=(left,),
                    device_id_type=pl.DeviceIdType.MESH,
                )
            xcp.wait()
            acc = comm[b] + local_partial()
            if s < N_DEV - 2:
                comm[b] = acc
            else:
                comm[b] = acc * jax.nn.sigmoid(acc)
                out_cp = pltpu.make_async_copy(comm.at[b], out_hbm, store_sem)
                out_cp.start()
                out_cp.wait()

    return pl.pallas_call(
        body,
        out_shape=jax.ShapeDtypeStruct((m_chunk, n), jnp.float32),
        in_specs=[
            pl.BlockSpec(memory_space=pltpu.ANY),
            pl.BlockSpec(memory_space=pltpu.VMEM),
        ],
        out_specs=pl.BlockSpec(memory_space=pltpu.ANY),
        scratch_shapes=[
            pltpu.VMEM((m_chunk, k_local), jnp.float32),
            pltpu.VMEM((2, m_chunk, n), jnp.float32),
            pltpu.SemaphoreType.DMA,
            pltpu.SemaphoreType.DMA,
            pltpu.SemaphoreType.DMA((2,)),
            pltpu.SemaphoreType.DMA((2,)),
            pltpu.SemaphoreType.REGULAR,
        ],
        compiler_params=pltpu.CompilerParams(collective_id=0),
    )(x, w_mat)


# baseline (device time: 1380461 ns/iter reference)
import jax
import jax.numpy as jnp
from jax import lax
from jax.experimental import pallas as pl
from jax.experimental.pallas import tpu as pltpu

N_DEV = 8


def kernel(x, w_mat):
    m, k_local = x.shape
    _, n = w_mat.shape
    m_chunk = m // N_DEV

    def body(x_hbm, w_ref, out_hbm, xbuf, comm, load_sem, store_sem,
             send_sems, recv_sems, credit_sem):
        my = lax.axis_index("i")
        left = lax.rem(my - 1 + N_DEV, N_DEV)
        right = lax.rem(my + 1, N_DEV)

        barrier = pltpu.get_barrier_semaphore()
        for nbr in (left, right):
            pl.semaphore_signal(
                barrier, inc=1, device_id=(nbr,),
                device_id_type=pl.DeviceIdType.MESH,
            )
        pl.semaphore_wait(barrier, 2)

        def load_chunk(c):
            cp = pltpu.make_async_copy(
                x_hbm.at[pl.ds(c * m_chunk, m_chunk), :], xbuf, load_sem)
            cp.start()
            return cp

        def local_partial():
            return jnp.dot(
                xbuf[...], w_ref[...], preferred_element_type=jnp.float32)

        load_chunk(lax.rem(my - 1 + N_DEV, N_DEV)).wait()
        comm[0] = local_partial()

        for s in range(N_DEV - 1):
            a = s % 2
            b = (s + 1) % 2
            if s >= 1:
                pl.semaphore_wait(credit_sem, 1)
            rdma = pltpu.make_async_remote_copy(
                src_ref=comm.at[a],
                dst_ref=comm.at[b],
                send_sem=send_sems.at[a],
                recv_sem=recv_sems.at[b],
                device_id=(right,),
                device_id_type=pl.DeviceIdType.MESH,
            )
            rdma.start()
            c = lax.rem(my - s - 2 + 2 * N_DEV, N_DEV)
            xcp = load_chunk(c)
            rdma.wait()
            if s < N_DEV - 2:
                pl.semaphore_signal(
                    credit_sem, inc=1, device_id=(left,),
                    device_id_type=pl.DeviceIdType.MESH,
                )
            xcp.wait()
            acc = comm[b] + local_partial()
            if s < N_DEV - 2:
                comm[b] = acc
            else:
                comm[b] = acc * jax.nn.sigmoid(acc)
                out_cp = pltpu.make_async_copy(comm.at[b], out_hbm, store_sem)
                out_cp.start()
                out_cp.wait()

    return pl.pallas_call(
        body,
        out_shape=jax.ShapeDtypeStruct((m_chunk, n), jnp.float32),
        in_specs=[
            pl.BlockSpec(memory_space=pl.ANY),
            pl.BlockSpec(memory_space=pltpu.VMEM),
        ],
        out_specs=pl.BlockSpec(memory_space=pl.ANY),
        scratch_shapes=[
            pltpu.VMEM((m_chunk, k_local), jnp.float32),
            pltpu.VMEM((2, m_chunk, n), jnp.float32),
            pltpu.SemaphoreType.DMA,
            pltpu.SemaphoreType.DMA,
            pltpu.SemaphoreType.DMA((2,)),
            pltpu.SemaphoreType.DMA((2,)),
            pltpu.SemaphoreType.REGULAR,
        ],
        compiler_params=pltpu.CompilerParams(
            collective_id=0, vmem_limit_bytes=100 * 1024 * 1024),
    )(x, w_mat)


# device time: 755760 ns/iter; 1.8266x vs baseline; 1.8266x over previous
import jax
import jax.numpy as jnp
from jax import lax
from jax.experimental import pallas as pl
from jax.experimental.pallas import tpu as pltpu

N_DEV = 8


def kernel(x, w_mat):
    m, k_local = x.shape
    _, n = w_mat.shape
    m_chunk = m // N_DEV
    nh = n // 2

    def body(x_hbm, w_ref, out_hbm, xbuf, cwb, ccwb, load_sems, store_sems,
             cw_send, cw_recv, ccw_send, ccw_recv, credit_cw, credit_ccw):
        my = lax.axis_index("i")
        left = lax.rem(my - 1 + N_DEV, N_DEV)
        right = lax.rem(my + 1, N_DEV)

        barrier = pltpu.get_barrier_semaphore()
        for nbr in (left, right):
            pl.semaphore_signal(
                barrier, inc=1, device_id=(nbr,),
                device_id_type=pl.DeviceIdType.MESH,
            )
        pl.semaphore_wait(barrier, 2)

        def load(c, slot):
            cp = pltpu.make_async_copy(
                x_hbm.at[pl.ds(c * m_chunk, m_chunk), :], xbuf.at[slot],
                load_sems.at[slot])
            cp.start()
            return cp

        def dot_half(slot, col0):
            return jnp.dot(
                xbuf[slot], w_ref[:, col0:col0 + nh],
                preferred_element_type=jnp.float32)

        l0 = load(lax.rem(my - 1 + N_DEV, N_DEV), 0)
        l1 = load(lax.rem(my + 1, N_DEV), 1)
        l0.wait()
        cwb[0] = dot_half(0, 0)
        l1.wait()
        ccwb[0] = dot_half(1, nh)

        for s in range(N_DEV - 1):
            a = s % 2
            b = (s + 1) % 2
            if s >= 1:
                pl.semaphore_wait(credit_cw, 1)
                pl.semaphore_wait(credit_ccw, 1)
            r_cw = pltpu.make_async_remote_copy(
                src_ref=cwb.at[a], dst_ref=cwb.at[b],
                send_sem=cw_send.at[a], recv_sem=cw_recv.at[b],
                device_id=(right,), device_id_type=pl.DeviceIdType.MESH,
            )
            r_ccw = pltpu.make_async_remote_copy(
                src_ref=ccwb.at[a], dst_ref=ccwb.at[b],
                send_sem=ccw_send.at[a], recv_sem=ccw_recv.at[b],
                device_id=(left,), device_id_type=pl.DeviceIdType.MESH,
            )
            r_cw.start()
            r_ccw.start()
            l0 = load(lax.rem(my - s - 2 + 2 * N_DEV, N_DEV), 0)
            l1 = load(lax.rem(my + s + 2, N_DEV), 1)
            r_cw.wait()
            r_ccw.wait()
            if s < N_DEV - 2:
                pl.semaphore_signal(
                    credit_cw, inc=1, device_id=(left,),
                    device_id_type=pl.DeviceIdType.MESH)
                pl.semaphore_signal(
                    credit_ccw, inc=1, device_id=(right,),
                    device_id_type=pl.DeviceIdType.MESH)
            l0.wait()
            if s < N_DEV - 2:
                cwb[b] = cwb[b] + dot_half(0, 0)
                l1.wait()
                ccwb[b] = ccwb[b] + dot_half(1, nh)
            else:
                acc_cw = cwb[b] + dot_half(0, 0)
                cwb[b] = acc_cw * jax.nn.sigmoid(acc_cw)
                l1.wait()
                acc_ccw = ccwb[b] + dot_half(1, nh)
                ccwb[b] = acc_ccw * jax.nn.sigmoid(acc_ccw)
                cp0 = pltpu.make_async_copy(
                    cwb.at[b], out_hbm.at[:, pl.ds(0, nh)], store_sems.at[0])
                cp1 = pltpu.make_async_copy(
                    ccwb.at[b], out_hbm.at[:, pl.ds(nh, nh)], store_sems.at[1])
                cp0.start()
                cp1.start()
                cp0.wait()
                cp1.wait()

    return pl.pallas_call(
        body,
        out_shape=jax.ShapeDtypeStruct((m_chunk, n), jnp.float32),
        in_specs=[
            pl.BlockSpec(memory_space=pl.ANY),
            pl.BlockSpec(memory_space=pltpu.VMEM),
        ],
        out_specs=pl.BlockSpec(memory_space=pl.ANY),
        scratch_shapes=[
            pltpu.VMEM((2, m_chunk, k_local), jnp.float32),
            pltpu.VMEM((2, m_chunk, nh), jnp.float32),
            pltpu.VMEM((2, m_chunk, nh), jnp.float32),
            pltpu.SemaphoreType.DMA((2,)),
            pltpu.SemaphoreType.DMA((2,)),
            pltpu.SemaphoreType.DMA((2,)),
            pltpu.SemaphoreType.DMA((2,)),
            pltpu.SemaphoreType.DMA((2,)),
            pltpu.SemaphoreType.DMA((2,)),
            pltpu.SemaphoreType.REGULAR,
            pltpu.SemaphoreType.REGULAR,
        ],
        compiler_params=pltpu.CompilerParams(
            collective_id=0, vmem_limit_bytes=100 * 1024 * 1024),
    )(x, w_mat)


# device time: 440816 ns/iter; 3.1316x vs baseline; 1.7145x over previous
import jax
import jax.numpy as jnp
from jax import lax
from jax.experimental import pallas as pl
from jax.experimental.pallas import tpu as pltpu

N_DEV = 8


def kernel(x, w_mat):
    m, k_local = x.shape
    _, n = w_mat.shape
    m_chunk = m // N_DEV
    nh = n // 2

    def body(x_hbm, w_ref, out_hbm, xbuf, cwb, ccwb, obuf, load_sems,
             store_sems, cw_send, cw_recv, ccw_send, ccw_recv,
             credit_cw, credit_ccw):
        my = lax.axis_index("i")
        left = lax.rem(my - 1 + N_DEV, N_DEV)
        right = lax.rem(my + 1, N_DEV)

        barrier = pltpu.get_barrier_semaphore()
        for nbr in (left, right):
            pl.semaphore_signal(
                barrier, inc=1, device_id=(nbr,),
                device_id_type=pl.DeviceIdType.MESH,
            )
        pl.semaphore_wait(barrier, 2)

        def load(c, slot):
            cp = pltpu.make_async_copy(
                x_hbm.at[pl.ds(c * m_chunk, m_chunk), :], xbuf.at[slot],
                load_sems.at[slot])
            cp.start()
            return cp

        def dot_half(slot, col0):
            return jnp.dot(
                xbuf[slot], w_ref[:, col0:col0 + nh],
                preferred_element_type=jnp.float32)

        l0 = load(lax.rem(my - 1 + N_DEV, N_DEV), 0)
        l1 = load(lax.rem(my + 1, N_DEV), 1)
        l0.wait()
        cwb[0] = dot_half(0, 0).astype(jnp.bfloat16)
        l1.wait()
        ccwb[0] = dot_half(1, nh).astype(jnp.bfloat16)

        for s in range(N_DEV - 1):
            a = s % 2
            b = (s + 1) % 2
            if s >= 1:
                pl.semaphore_wait(credit_cw, 1)
                pl.semaphore_wait(credit_ccw, 1)
            r_cw = pltpu.make_async_remote_copy(
                src_ref=cwb.at[a], dst_ref=cwb.at[b],
                send_sem=cw_send.at[a], recv_sem=cw_recv.at[b],
                device_id=(right,), device_id_type=pl.DeviceIdType.MESH,
            )
            r_ccw = pltpu.make_async_remote_copy(
                src_ref=ccwb.at[a], dst_ref=ccwb.at[b],
                send_sem=ccw_send.at[a], recv_sem=ccw_recv.at[b],
                device_id=(left,), device_id_type=pl.DeviceIdType.MESH,
            )
            r_cw.start()
            r_ccw.start()
            l0 = load(lax.rem(my - s - 2 + 2 * N_DEV, N_DEV), 0)
            l1 = load(lax.rem(my + s + 2, N_DEV), 1)
            r_cw.wait()
            r_ccw.wait()
            if s < N_DEV - 2:
                pl.semaphore_signal(
                    credit_cw, inc=1, device_id=(left,),
                    device_id_type=pl.DeviceIdType.MESH)
                pl.semaphore_signal(
                    credit_ccw, inc=1, device_id=(right,),
                    device_id_type=pl.DeviceIdType.MESH)
            l0.wait()
            if s < N_DEV - 2:
                cwb[b] = (cwb[b].astype(jnp.float32)
                          + dot_half(0, 0)).astype(jnp.bfloat16)
                l1.wait()
                ccwb[b] = (ccwb[b].astype(jnp.float32)
                           + dot_half(1, nh)).astype(jnp.bfloat16)
            else:
                acc_cw = cwb[b].astype(jnp.float32) + dot_half(0, 0)
                obuf[0] = acc_cw * jax.nn.sigmoid(acc_cw)
                l1.wait()
                acc_ccw = ccwb[b].astype(jnp.float32) + dot_half(1, nh)
                obuf[1] = acc_ccw * jax.nn.sigmoid(acc_ccw)
                cp0 = pltpu.make_async_copy(
                    obuf.at[0], out_hbm.at[:, pl.ds(0, nh)], store_sems.at[0])
                cp1 = pltpu.make_async_copy(
                    obuf.at[1], out_hbm.at[:, pl.ds(nh, nh)], store_sems.at[1])
                cp0.start()
                cp1.start()
                cp0.wait()
                cp1.wait()

    return pl.pallas_call(
        body,
        out_shape=jax.ShapeDtypeStruct((m_chunk, n), jnp.float32),
        in_specs=[
            pl.BlockSpec(memory_space=pl.ANY),
            pl.BlockSpec(memory_space=pltpu.VMEM),
        ],
        out_specs=pl.BlockSpec(memory_space=pl.ANY),
        scratch_shapes=[
            pltpu.VMEM((2, m_chunk, k_local), jnp.float32),
            pltpu.VMEM((2, m_chunk, nh), jnp.bfloat16),
            pltpu.VMEM((2, m_chunk, nh), jnp.bfloat16),
            pltpu.VMEM((2, m_chunk, nh), jnp.float32),
            pltpu.SemaphoreType.DMA((2,)),
            pltpu.SemaphoreType.DMA((2,)),
            pltpu.SemaphoreType.DMA((2,)),
            pltpu.SemaphoreType.DMA((2,)),
            pltpu.SemaphoreType.DMA((2,)),
            pltpu.SemaphoreType.DMA((2,)),
            pltpu.SemaphoreType.REGULAR,
            pltpu.SemaphoreType.REGULAR,
        ],
        compiler_params=pltpu.CompilerParams(
            collective_id=0, vmem_limit_bytes=100 * 1024 * 1024),
    )(x, w_mat)


# device time: 414395 ns/iter; 3.3313x vs baseline; 1.0638x over previous
import jax
import jax.numpy as jnp
from jax import lax
from jax.experimental import pallas as pl
from jax.experimental.pallas import tpu as pltpu

N_DEV = 8
N_SEG = 2


def kernel(x, w_mat):
    m, k_local = x.shape
    _, n = w_mat.shape
    m_chunk = m // N_DEV
    nh = n // 2
    segw = nh // N_SEG

    def body(x_hbm, w_ref, out_hbm, xbuf, cwb, ccwb, obuf, load_sems,
             store_sems, cw_send, cw_recv, ccw_send, ccw_recv,
             credit_cw, credit_ccw):
        my = lax.axis_index("i")
        left = lax.rem(my - 1 + N_DEV, N_DEV)
        right = lax.rem(my + 1, N_DEV)

        barrier = pltpu.get_barrier_semaphore()
        for nbr in (left, right):
            pl.semaphore_signal(
                barrier, inc=1, device_id=(nbr,),
                device_id_type=pl.DeviceIdType.MESH,
            )
        pl.semaphore_wait(barrier, 2)

        def load(c, slot):
            cp = pltpu.make_async_copy(
                x_hbm.at[pl.ds(c * m_chunk, m_chunk), :], xbuf.at[slot],
                load_sems.at[slot])
            cp.start()
            return cp

        def dot_seg(slot, col0):
            return jnp.dot(
                xbuf[slot], w_ref[:, col0:col0 + segw],
                preferred_element_type=jnp.float32)

        l0 = load(lax.rem(my - 1 + N_DEV, N_DEV), 0)
        l1 = load(lax.rem(my + 1, N_DEV), 1)
        l0.wait()
        for g in range(N_SEG):
            cwb[0, g] = dot_seg(0, g * segw).astype(jnp.bfloat16)
        l1.wait()
        for g in range(N_SEG):
            ccwb[0, g] = dot_seg(1, nh + g * segw).astype(jnp.bfloat16)

        for s in range(N_DEV - 1):
            a = s % 2
            b = (s + 1) % 2
            last = s == N_DEV - 2
            if s >= 1:
                pl.semaphore_wait(credit_cw, 1)
                pl.semaphore_wait(credit_ccw, 1)
            r_cw = []
            r_ccw = []
            for g in range(N_SEG):
                r = pltpu.make_async_remote_copy(
                    src_ref=cwb.at[a, g], dst_ref=cwb.at[b, g],
                    send_sem=cw_send.at[a, g], recv_sem=cw_recv.at[b, g],
                    device_id=(right,), device_id_type=pl.DeviceIdType.MESH,
                )
                r.start()
                r_cw.append(r)
            for g in range(N_SEG):
                r = pltpu.make_async_remote_copy(
                    src_ref=ccwb.at[a, g], dst_ref=ccwb.at[b, g],
                    send_sem=ccw_send.at[a, g], recv_sem=ccw_recv.at[b, g],
                    device_id=(left,), device_id_type=pl.DeviceIdType.MESH,
                )
                r.start()
                r_ccw.append(r)
            l0 = load(lax.rem(my - s - 2 + 2 * N_DEV, N_DEV), 0)
            l1 = load(lax.rem(my + s + 2, N_DEV), 1)
            l0.wait()
            l1.wait()
            for g in range(N_SEG):
                r_cw[g].wait()
                if not last:
                    cwb[b, g] = (cwb[b, g].astype(jnp.float32)
                                 + dot_seg(0, g * segw)).astype(jnp.bfloat16)
                else:
                    acc = cwb[b, g].astype(jnp.float32) + dot_seg(0, g * segw)
                    obuf[0, :, g * segw:(g + 1) * segw] = (
                        acc * jax.nn.sigmoid(acc))
                r_ccw[g].wait()
                if not last:
                    ccwb[b, g] = (ccwb[b, g].astype(jnp.float32)
                                  + dot_seg(1, nh + g * segw)
                                  ).astype(jnp.bfloat16)
                else:
                    acc = (ccwb[b, g].astype(jnp.float32)
                           + dot_seg(1, nh + g * segw))
                    obuf[1, :, g * segw:(g + 1) * segw] = (
                        acc * jax.nn.sigmoid(acc))
            if not last:
                pl.semaphore_signal(
                    credit_cw, inc=1, device_id=(left,),
                    device_id_type=pl.DeviceIdType.MESH)
                pl.semaphore_signal(
                    credit_ccw, inc=1, device_id=(right,),
                    device_id_type=pl.DeviceIdType.MESH)
            else:
                cp0 = pltpu.make_async_copy(
                    obuf.at[0], out_hbm.at[:, pl.ds(0, nh)], store_sems.at[0])
                cp1 = pltpu.make_async_copy(
                    obuf.at[1], out_hbm.at[:, pl.ds(nh, nh)], store_sems.at[1])
                cp0.start()
                cp1.start()
                cp0.wait()
                cp1.wait()

    return pl.pallas_call(
        body,
        out_shape=jax.ShapeDtypeStruct((m_chunk, n), jnp.float32),
        in_specs=[
            pl.BlockSpec(memory_space=pl.ANY),
            pl.BlockSpec(memory_space=pltpu.VMEM),
        ],
        out_specs=pl.BlockSpec(memory_space=pl.ANY),
        scratch_shapes=[
            pltpu.VMEM((2, m_chunk, k_local), jnp.float32),
            pltpu.VMEM((2, N_SEG, m_chunk, segw), jnp.bfloat16),
            pltpu.VMEM((2, N_SEG, m_chunk, segw), jnp.bfloat16),
            pltpu.VMEM((2, m_chunk, nh), jnp.float32),
            pltpu.SemaphoreType.DMA((2,)),
            pltpu.SemaphoreType.DMA((2,)),
            pltpu.SemaphoreType.DMA((2, N_SEG)),
            pltpu.SemaphoreType.DMA((2, N_SEG)),
            pltpu.SemaphoreType.DMA((2, N_SEG)),
            pltpu.SemaphoreType.DMA((2, N_SEG)),
            pltpu.SemaphoreType.REGULAR,
            pltpu.SemaphoreType.REGULAR,
        ],
        compiler_params=pltpu.CompilerParams(
            collective_id=0, vmem_limit_bytes=100 * 1024 * 1024),
    )(x, w_mat)


# device time: 397022 ns/iter; 3.4770x vs baseline; 1.0438x over previous
import jax
import jax.numpy as jnp
from jax import lax
from jax.experimental import pallas as pl
from jax.experimental.pallas import tpu as pltpu

N_DEV = 8
N_SEG = 4


def kernel(x, w_mat):
    m, k_local = x.shape
    _, n = w_mat.shape
    m_chunk = m // N_DEV
    nh = n // 2
    segw = nh // N_SEG

    def body(x_hbm, w_ref, out_hbm, xbuf, cwb, ccwb, obuf, load_sems,
             store_sems, cw_send, cw_recv, ccw_send, ccw_recv,
             credit_cw, credit_ccw):
        my = lax.axis_index("i")
        left = lax.rem(my - 1 + N_DEV, N_DEV)
        right = lax.rem(my + 1, N_DEV)

        barrier = pltpu.get_barrier_semaphore()
        for nbr in (left, right):
            pl.semaphore_signal(
                barrier, inc=1, device_id=(nbr,),
                device_id_type=pl.DeviceIdType.MESH,
            )
        pl.semaphore_wait(barrier, 2)

        def load(c, slot):
            cp = pltpu.make_async_copy(
                x_hbm.at[pl.ds(c * m_chunk, m_chunk), :], xbuf.at[slot],
                load_sems.at[slot])
            cp.start()
            return cp

        def dot_seg(slot, col0):
            return jnp.dot(
                xbuf[slot], w_ref[:, col0:col0 + segw],
                preferred_element_type=jnp.float32)

        l0 = load(lax.rem(my - 1 + N_DEV, N_DEV), 0)
        l1 = load(lax.rem(my + 1, N_DEV), 1)
        l0.wait()
        for g in range(N_SEG):
            cwb[0, g] = dot_seg(0, g * segw).astype(jnp.bfloat16)
        l1.wait()
        for g in range(N_SEG):
            ccwb[0, g] = dot_seg(1, nh + g * segw).astype(jnp.bfloat16)

        for s in range(N_DEV - 1):
            a = s % 2
            b = (s + 1) % 2
            last = s == N_DEV - 2
            if s >= 1:
                pl.semaphore_wait(credit_cw, 1)
                pl.semaphore_wait(credit_ccw, 1)
            r_cw = []
            r_ccw = []
            for g in range(N_SEG):
                r = pltpu.make_async_remote_copy(
                    src_ref=cwb.at[a, g], dst_ref=cwb.at[b, g],
                    send_sem=cw_send.at[a, g], recv_sem=cw_recv.at[b, g],
                    device_id=(right,), device_id_type=pl.DeviceIdType.MESH,
                )
                r.start()
                r_cw.append(r)
            for g in range(N_SEG):
                r = pltpu.make_async_remote_copy(
                    src_ref=ccwb.at[a, g], dst_ref=ccwb.at[b, g],
                    send_sem=ccw_send.at[a, g], recv_sem=ccw_recv.at[b, g],
                    device_id=(left,), device_id_type=pl.DeviceIdType.MESH,
                )
                r.start()
                r_ccw.append(r)
            l0 = load(lax.rem(my - s - 2 + 2 * N_DEV, N_DEV), 0)
            l1 = load(lax.rem(my + s + 2, N_DEV), 1)
            l0.wait()
            l1.wait()
            for g in range(N_SEG):
                r_cw[g].wait()
                if not last:
                    cwb[b, g] = (cwb[b, g].astype(jnp.float32)
                                 + dot_seg(0, g * segw)).astype(jnp.bfloat16)
                else:
                    acc = cwb[b, g].astype(jnp.float32) + dot_seg(0, g * segw)
                    obuf[0, :, g * segw:(g + 1) * segw] = (
                        acc * jax.nn.sigmoid(acc))
                r_ccw[g].wait()
                if not last:
                    ccwb[b, g] = (ccwb[b, g].astype(jnp.float32)
                                  + dot_seg(1, nh + g * segw)
                                  ).astype(jnp.bfloat16)
                else:
                    acc = (ccwb[b, g].astype(jnp.float32)
                           + dot_seg(1, nh + g * segw))
                    obuf[1, :, g * segw:(g + 1) * segw] = (
                        acc * jax.nn.sigmoid(acc))
            if not last:
                pl.semaphore_signal(
                    credit_cw, inc=1, device_id=(left,),
                    device_id_type=pl.DeviceIdType.MESH)
                pl.semaphore_signal(
                    credit_ccw, inc=1, device_id=(right,),
                    device_id_type=pl.DeviceIdType.MESH)
            else:
                cp0 = pltpu.make_async_copy(
                    obuf.at[0], out_hbm.at[:, pl.ds(0, nh)], store_sems.at[0])
                cp1 = pltpu.make_async_copy(
                    obuf.at[1], out_hbm.at[:, pl.ds(nh, nh)], store_sems.at[1])
                cp0.start()
                cp1.start()
                cp0.wait()
                cp1.wait()

    return pl.pallas_call(
        body,
        out_shape=jax.ShapeDtypeStruct((m_chunk, n), jnp.float32),
        in_specs=[
            pl.BlockSpec(memory_space=pl.ANY),
            pl.BlockSpec(memory_space=pltpu.VMEM),
        ],
        out_specs=pl.BlockSpec(memory_space=pl.ANY),
        scratch_shapes=[
            pltpu.VMEM((2, m_chunk, k_local), jnp.float32),
            pltpu.VMEM((2, N_SEG, m_chunk, segw), jnp.bfloat16),
            pltpu.VMEM((2, N_SEG, m_chunk, segw), jnp.bfloat16),
            pltpu.VMEM((2, m_chunk, nh), jnp.float32),
            pltpu.SemaphoreType.DMA((2,)),
            pltpu.SemaphoreType.DMA((2,)),
            pltpu.SemaphoreType.DMA((2, N_SEG)),
            pltpu.SemaphoreType.DMA((2, N_SEG)),
            pltpu.SemaphoreType.DMA((2, N_SEG)),
            pltpu.SemaphoreType.DMA((2, N_SEG)),
            pltpu.SemaphoreType.REGULAR,
            pltpu.SemaphoreType.REGULAR,
        ],
        compiler_params=pltpu.CompilerParams(
            collective_id=0, vmem_limit_bytes=100 * 1024 * 1024),
    )(x, w_mat)


# device time: 391980 ns/iter; 3.5218x vs baseline; 1.0129x over previous
import jax
import jax.numpy as jnp
from jax import lax
from jax.experimental import pallas as pl
from jax.experimental.pallas import tpu as pltpu

N_DEV = 8
N_SEG = 4


def kernel(x, w_mat):
    m, k_local = x.shape
    _, n = w_mat.shape
    m_chunk = m // N_DEV
    nh = n // 2
    segw = nh // N_SEG

    def body(x_hbm, w_ref, out_hbm, xbuf, cwb, ccwb, obuf, load_sems,
             store_sems, cw_send, cw_recv, ccw_send, ccw_recv,
             credit_cw, credit_ccw):
        my = lax.axis_index("i")
        left = lax.rem(my - 1 + N_DEV, N_DEV)
        right = lax.rem(my + 1, N_DEV)

        barrier = pltpu.get_barrier_semaphore()
        for nbr in (left, right):
            pl.semaphore_signal(
                barrier, inc=1, device_id=(nbr,),
                device_id_type=pl.DeviceIdType.MESH,
            )
        pl.semaphore_wait(barrier, 2)

        def load(c, slot):
            cp = pltpu.make_async_copy(
                x_hbm.at[pl.ds(c * m_chunk, m_chunk), :], xbuf.at[slot],
                load_sems.at[slot])
            cp.start()
            return cp

        def dot_seg(slot, col0):
            return jnp.dot(
                xbuf[slot], w_ref[:, col0:col0 + segw],
                preferred_element_type=jnp.float32)

        l0 = load(lax.rem(my - 1 + N_DEV, N_DEV), 0)
        l1 = load(lax.rem(my + 1, N_DEV), 1)
        r_cw0 = []
        r_ccw0 = []
        l0.wait()
        for g in range(N_SEG):
            cwb[0, g] = dot_seg(0, g * segw).astype(jnp.bfloat16)
            r = pltpu.make_async_remote_copy(
                src_ref=cwb.at[0, g], dst_ref=cwb.at[1, g],
                send_sem=cw_send.at[0, g], recv_sem=cw_recv.at[1, g],
                device_id=(right,), device_id_type=pl.DeviceIdType.MESH,
            )
            r.start()
            r_cw0.append(r)
        l1.wait()
        for g in range(N_SEG):
            ccwb[0, g] = dot_seg(1, nh + g * segw).astype(jnp.bfloat16)
            r = pltpu.make_async_remote_copy(
                src_ref=ccwb.at[0, g], dst_ref=ccwb.at[1, g],
                send_sem=ccw_send.at[0, g], recv_sem=ccw_recv.at[1, g],
                device_id=(left,), device_id_type=pl.DeviceIdType.MESH,
            )
            r.start()
            r_ccw0.append(r)

        for s in range(N_DEV - 1):
            a = s % 2
            b = (s + 1) % 2
            last = s == N_DEV - 2
            if s == 0:
                r_cw = r_cw0
                r_ccw = r_ccw0
            else:
                pl.semaphore_wait(credit_cw, 1)
                pl.semaphore_wait(credit_ccw, 1)
                r_cw = []
                r_ccw = []
                for g in range(N_SEG):
                    r = pltpu.make_async_remote_copy(
                        src_ref=cwb.at[a, g], dst_ref=cwb.at[b, g],
                        send_sem=cw_send.at[a, g], recv_sem=cw_recv.at[b, g],
                        device_id=(right,),
                        device_id_type=pl.DeviceIdType.MESH,
                    )
                    r.start()
                    r_cw.append(r)
                for g in range(N_SEG):
                    r = pltpu.make_async_remote_copy(
                        src_ref=ccwb.at[a, g], dst_ref=ccwb.at[b, g],
                        send_sem=ccw_send.at[a, g], recv_sem=ccw_recv.at[b, g],
                        device_id=(left,),
                        device_id_type=pl.DeviceIdType.MESH,
                    )
                    r.start()
                    r_ccw.append(r)
            l0 = load(lax.rem(my - s - 2 + 2 * N_DEV, N_DEV), 0)
            l1 = load(lax.rem(my + s + 2, N_DEV), 1)
            l0.wait()
            l1.wait()
            for g in range(N_SEG):
                r_cw[g].wait()
                if not last:
                    cwb[b, g] = (cwb[b, g].astype(jnp.float32)
                                 + dot_seg(0, g * segw)).astype(jnp.bfloat16)
                else:
                    acc = cwb[b, g].astype(jnp.float32) + dot_seg(0, g * segw)
                    obuf[0, :, g * segw:(g + 1) * segw] = (
                        acc * jax.nn.sigmoid(acc))
                r_ccw[g].wait()
                if not last:
                    ccwb[b, g] = (ccwb[b, g].astype(jnp.float32)
                                  + dot_seg(1, nh + g * segw)
                                  ).astype(jnp.bfloat16)
                else:
                    acc = (ccwb[b, g].astype(jnp.float32)
                           + dot_seg(1, nh + g * segw))
                    obuf[1, :, g * segw:(g + 1) * segw] = (
                        acc * jax.nn.sigmoid(acc))
            if not last:
                pl.semaphore_signal(
                    credit_cw, inc=1, device_id=(left,),
                    device_id_type=pl.DeviceIdType.MESH)
                pl.semaphore_signal(
                    credit_ccw, inc=1, device_id=(right,),
                    device_id_type=pl.DeviceIdType.MESH)
            else:
                cp0 = pltpu.make_async_copy(
                    obuf.at[0], out_hbm.at[:, pl.ds(0, nh)], store_sems.at[0])
                cp1 = pltpu.make_async_copy(
                    obuf.at[1], out_hbm.at[:, pl.ds(nh, nh)], store_sems.at[1])
                cp0.start()
                cp1.start()
                cp0.wait()
                cp1.wait()

    return pl.pallas_call(
        body,
        out_shape=jax.ShapeDtypeStruct((m_chunk, n), jnp.float32),
        in_specs=[
            pl.BlockSpec(memory_space=pl.ANY),
            pl.BlockSpec(memory_space=pltpu.VMEM),
        ],
        out_specs=pl.BlockSpec(memory_space=pl.ANY),
        scratch_shapes=[
            pltpu.VMEM((2, m_chunk, k_local), jnp.float32),
            pltpu.VMEM((2, N_SEG, m_chunk, segw), jnp.bfloat16),
            pltpu.VMEM((2, N_SEG, m_chunk, segw), jnp.bfloat16),
            pltpu.VMEM((2, m_chunk, nh), jnp.float32),
            pltpu.SemaphoreType.DMA((2,)),
            pltpu.SemaphoreType.DMA((2,)),
            pltpu.SemaphoreType.DMA((2, N_SEG)),
            pltpu.SemaphoreType.DMA((2, N_SEG)),
            pltpu.SemaphoreType.DMA((2, N_SEG)),
            pltpu.SemaphoreType.DMA((2, N_SEG)),
            pltpu.SemaphoreType.REGULAR,
            pltpu.SemaphoreType.REGULAR,
        ],
        compiler_params=pltpu.CompilerParams(
            collective_id=0, vmem_limit_bytes=100 * 1024 * 1024),
    )(x, w_mat)


# device time: 355416 ns/iter; 3.8841x vs baseline; 1.1029x over previous
import jax
import jax.numpy as jnp
from jax import lax
from jax.experimental import pallas as pl
from jax.experimental.pallas import tpu as pltpu

N_DEV = 8
N_SEG = 4


def kernel(x, w_mat):
    m, k_local = x.shape
    _, n = w_mat.shape
    m_chunk = m // N_DEV
    nh = n // 2
    segw = nh // N_SEG

    def body(x_hbm, w_ref, out_hbm, xbuf, cwb, ccwb, obuf, load_sems,
             store_sems, cw_send, cw_recv, ccw_send, ccw_recv,
             credit_cw, credit_ccw):
        my = lax.axis_index("i")
        left = lax.rem(my - 1 + N_DEV, N_DEV)
        right = lax.rem(my + 1, N_DEV)

        barrier = pltpu.get_barrier_semaphore()
        for nbr in (left, right):
            pl.semaphore_signal(
                barrier, inc=1, device_id=(nbr,),
                device_id_type=pl.DeviceIdType.MESH,
            )
        pl.semaphore_wait(barrier, 2)

        def load(c, slot):
            cp = pltpu.make_async_copy(
                x_hbm.at[pl.ds(c * m_chunk, m_chunk), :], xbuf.at[slot],
                load_sems.at[slot])
            cp.start()
            return cp

        def dot_seg(slot, col0):
            return jnp.dot(
                xbuf[slot], w_ref[:, col0:col0 + segw],
                preferred_element_type=jnp.float32)

        def rc(buf, sl_src, sl_dst, g, send, recv, dst):
            return pltpu.make_async_remote_copy(
                src_ref=buf.at[sl_src, g], dst_ref=buf.at[sl_dst, g],
                send_sem=send.at[sl_src, g], recv_sem=recv.at[sl_dst, g],
                device_id=(dst,), device_id_type=pl.DeviceIdType.MESH,
            )

        l0 = load(lax.rem(my - 1 + N_DEV, N_DEV), 0)
        l1 = load(lax.rem(my + 1, N_DEV), 1)
        r_cw = []
        r_ccw = []
        l0.wait()
        for g in range(N_SEG):
            cwb[0, g] = dot_seg(0, g * segw).astype(jnp.bfloat16)
            r = rc(cwb, 0, 1, g, cw_send, cw_recv, right)
            r.start()
            r_cw.append(r)
        l1.wait()
        for g in range(N_SEG):
            ccwb[0, g] = dot_seg(1, nh + g * segw).astype(jnp.bfloat16)
            r = rc(ccwb, 0, 1, g, ccw_send, ccw_recv, left)
            r.start()
            r_ccw.append(r)

        for t in range(1, N_DEV):
            q = t % 2
            p = 1 - q
            last = t == N_DEV - 1
            l0 = load(lax.rem(my - t - 1 + 2 * N_DEV, N_DEV), 0)
            l1 = load(lax.rem(my + t + 1, N_DEV), 1)
            new_cw = []
            new_ccw = []
            for g in range(N_SEG):
                col_cw = g * segw
                col_ccw = nh + g * segw
                r_cw[g].wait()
                if not last:
                    pl.semaphore_signal(
                        credit_cw, inc=1, device_id=(left,),
                        device_id_type=pl.DeviceIdType.MESH)
                if g == 0:
                    l0.wait()
                if not last:
                    cwb[q, g] = (cwb[q, g].astype(jnp.float32)
                                 + dot_seg(0, col_cw)).astype(jnp.bfloat16)
                    pl.semaphore_wait(credit_cw, 1)
                    r = rc(cwb, q, p, g, cw_send, cw_recv, right)
                    r.start()
                    new_cw.append(r)
                else:
                    acc = cwb[q, g].astype(jnp.float32) + dot_seg(0, col_cw)
                    obuf[0, :, col_cw:col_cw + segw] = (
                        acc * jax.nn.sigmoid(acc))
                r_ccw[g].wait()
                if not last:
                    pl.semaphore_signal(
                        credit_ccw, inc=1, device_id=(right,),
                        device_id_type=pl.DeviceIdType.MESH)
                if g == 0:
                    l1.wait()
                if not last:
                    ccwb[q, g] = (ccwb[q, g].astype(jnp.float32)
                                  + dot_seg(1, col_ccw)).astype(jnp.bfloat16)
                    pl.semaphore_wait(credit_ccw, 1)
                    r = rc(ccwb, q, p, g, ccw_send, ccw_recv, left)
                    r.start()
                    new_ccw.append(r)
                else:
                    acc = (ccwb[q, g].astype(jnp.float32)
                           + dot_seg(1, col_ccw))
                    obuf[1, :, g * segw:(g + 1) * segw] = (
                        acc * jax.nn.sigmoid(acc))
            r_cw = new_cw
            r_ccw = new_ccw

        cp0 = pltpu.make_async_copy(
            obuf.at[0], out_hbm.at[:, pl.ds(0, nh)], store_sems.at[0])
        cp1 = pltpu.make_async_copy(
            obuf.at[1], out_hbm.at[:, pl.ds(nh, nh)], store_sems.at[1])
        cp0.start()
        cp1.start()
        cp0.wait()
        cp1.wait()

    return pl.pallas_call(
        body,
        out_shape=jax.ShapeDtypeStruct((m_chunk, n), jnp.float32),
        in_specs=[
            pl.BlockSpec(memory_space=pl.ANY),
            pl.BlockSpec(memory_space=pltpu.VMEM),
        ],
        out_specs=pl.BlockSpec(memory_space=pl.ANY),
        scratch_shapes=[
            pltpu.VMEM((2, m_chunk, k_local), jnp.float32),
            pltpu.VMEM((2, N_SEG, m_chunk, segw), jnp.bfloat16),
            pltpu.VMEM((2, N_SEG, m_chunk, segw), jnp.bfloat16),
            pltpu.VMEM((2, m_chunk, nh), jnp.float32),
            pltpu.SemaphoreType.DMA((2,)),
            pltpu.SemaphoreType.DMA((2,)),
            pltpu.SemaphoreType.DMA((2, N_SEG)),
            pltpu.SemaphoreType.DMA((2, N_SEG)),
            pltpu.SemaphoreType.DMA((2, N_SEG)),
            pltpu.SemaphoreType.DMA((2, N_SEG)),
            pltpu.SemaphoreType.REGULAR,
            pltpu.SemaphoreType.REGULAR,
        ],
        compiler_params=pltpu.CompilerParams(
            collective_id=0, vmem_limit_bytes=100 * 1024 * 1024),
    )(x, w_mat)
